# baseline (device time: 191879 ns/iter reference)
import jax
import jax.numpy as jnp
from jax import lax
from jax.experimental import pallas as pl
from jax.experimental.pallas import tpu as pltpu

N_DEV = 8
M = 2048
N = 2048

MASKS = (1, 3, 4)


def kernel(A, B):
    a16 = A.astype(jnp.bfloat16)
    b16 = B.astype(jnp.bfloat16)

    def body(
        a_ref,
        b_ref,
        out_ref,
        r0,
        r1,
        r2,
        rs_send_sems,
        rs_recv_sems,
        ag_send_sems,
        ag_recv_sems,
    ):
        my = lax.axis_index("i")
        b0 = my & 1
        b1 = (my >> 1) & 1
        b2 = (my >> 2) & 1
        sides = (b0 ^ b1, b0, b2)
        recv_bufs = (r0, r1, r2)

        barrier_sem = pltpu.get_barrier_semaphore()
        for m in MASKS:
            pl.semaphore_signal(
                barrier_sem,
                inc=1,
                device_id=(my ^ m,),
                device_id_type=pl.DeviceIdType.MESH,
            )
        pl.semaphore_wait(barrier_sem, 3)

        out_ref[:, :] = jnp.dot(
            a_ref[:, :], b_ref[:, :], preferred_element_type=jnp.float32
        ).astype(jnp.bfloat16)

        base = 0 * my
        length = M
        for t in range(3):
            length //= 2
            side = sides[t]
            keep = base + side * length
            send = base + (1 - side) * length
            rdma = pltpu.make_async_remote_copy(
                src_ref=out_ref.at[pl.ds(send, length), :],
                dst_ref=recv_bufs[t],
                send_sem=rs_send_sems.at[t],
                recv_sem=rs_recv_sems.at[t],
                device_id=(my ^ MASKS[t],),
                device_id_type=pl.DeviceIdType.MESH,
            )
            rdma.start()
            rdma.wait()
            acc = out_ref[pl.ds(keep, length), :].astype(jnp.float32) + recv_bufs[
                t
            ][:, :].astype(jnp.float32)
            out_ref[pl.ds(keep, length), :] = acc.astype(jnp.bfloat16)
            base = keep

        for t in (2, 1, 0):
            length = M >> (t + 1)
            rdma = pltpu.make_async_remote_copy(
                src_ref=out_ref.at[pl.ds(base, length), :],
                dst_ref=out_ref.at[pl.ds(base, length), :],
                send_sem=ag_send_sems.at[t],
                recv_sem=ag_recv_sems.at[t],
                device_id=(my ^ MASKS[t],),
                device_id_type=pl.DeviceIdType.MESH,
            )
            rdma.start()
            rdma.wait()
            base = base - sides[t] * length

    return pl.pallas_call(
        body,
        out_shape=jax.ShapeDtypeStruct((M, N), jnp.bfloat16),
        in_specs=[
            pl.BlockSpec(memory_space=pltpu.VMEM),
            pl.BlockSpec(memory_space=pltpu.VMEM),
        ],
        out_specs=pl.BlockSpec(memory_space=pltpu.VMEM),
        scratch_shapes=[
            pltpu.VMEM((M // 2, N), jnp.bfloat16),
            pltpu.VMEM((M // 4, N), jnp.bfloat16),
            pltpu.VMEM((M // 8, N), jnp.bfloat16),
            pltpu.SemaphoreType.DMA((3,)),
            pltpu.SemaphoreType.DMA((3,)),
            pltpu.SemaphoreType.DMA((3,)),
            pltpu.SemaphoreType.DMA((3,)),
        ],
        compiler_params=pltpu.CompilerParams(collective_id=0),
    )(a16, b16)


# device time: 88032 ns/iter; 2.1797x vs baseline; 2.1797x over previous
import jax
import jax.numpy as jnp
from jax import lax
from jax.experimental import pallas as pl
from jax.experimental.pallas import tpu as pltpu

N_DEV = 8
M = 2048
N = 2048

GROUPS = ((0, 768), (768, 640), (1408, 640))

GMASKS = ((1, 3, 4), (3, 4, 1), (4, 1, 3))


def kernel(A, B):
    a16 = A.astype(jnp.bfloat16)
    b16 = B.astype(jnp.bfloat16)

    def body(a_ref, b_ref, out_ref, *scratch):
        rbufs = [scratch[3 * g : 3 * g + 3] for g in range(3)]
        rs_ssem, rs_rsem, ag_ssem, ag_rsem = scratch[9:]

        my = lax.axis_index("i")
        b0 = my & 1
        b1 = (my >> 1) & 1
        b2 = (my >> 2) & 1
        gsides = (
            (b0 ^ b1, b0, b2),
            (b1, b2, b0),
            (b2, b0 ^ b1, b0),
        )

        barrier_sem = pltpu.get_barrier_semaphore()
        for m in (1, 3, 4):
            pl.semaphore_signal(
                barrier_sem,
                inc=1,
                device_id=(my ^ m,),
                device_id_type=pl.DeviceIdType.MESH,
            )
        pl.semaphore_wait(barrier_sem, 3)

        bases = [None, None, None]
        pend = [None, None, None]
        done = []

        def start_rs(g, t):
            glen = GROUPS[g][1]
            length = glen >> (t + 1)
            side = gsides[g][t]
            keep = bases[g] + side * length
            send = bases[g] + (1 - side) * length
            rdma = pltpu.make_async_remote_copy(
                src_ref=out_ref.at[pl.ds(send, length), :],
                dst_ref=rbufs[g][t],
                send_sem=rs_ssem.at[3 * g + t],
                recv_sem=rs_rsem.at[3 * g + t],
                device_id=(my ^ GMASKS[g][t],),
                device_id_type=pl.DeviceIdType.MESH,
            )
            rdma.start()
            pend[g] = (rdma, keep)

        def start_ag(g, t):
            length = GROUPS[g][1] >> (t + 1)
            rdma = pltpu.make_async_remote_copy(
                src_ref=out_ref.at[pl.ds(bases[g], length), :],
                dst_ref=out_ref.at[pl.ds(bases[g], length), :],
                send_sem=ag_ssem.at[3 * g + t],
                recv_sem=ag_rsem.at[3 * g + t],
                device_id=(my ^ GMASKS[g][t],),
                device_id_type=pl.DeviceIdType.MESH,
            )
            rdma.start()
            pend[g] = rdma

        for g, (off, glen) in enumerate(GROUPS):
            out_ref[pl.ds(off, glen), :] = jnp.dot(
                a_ref[pl.ds(off, glen), :],
                b_ref[:, :],
                preferred_element_type=jnp.float32,
            ).astype(jnp.bfloat16)
            bases[g] = off + 0 * my
            start_rs(g, 0)

        for t in range(3):
            for g in range(3):
                rdma, keep = pend[g]
                rdma.wait_recv()
                done.append(rdma)
                length = GROUPS[g][1] >> (t + 1)
                acc = out_ref[pl.ds(keep, length), :].astype(jnp.float32) + rbufs[
                    g
                ][t][:, :].astype(jnp.float32)
                out_ref[pl.ds(keep, length), :] = acc.astype(jnp.bfloat16)
                bases[g] = keep
                if t < 2:
                    start_rs(g, t + 1)

        for g in range(3):
            start_ag(g, 2)
        for t in (2, 1, 0):
            for g in range(3):
                rdma = pend[g]
                rdma.wait_recv()
                done.append(rdma)
                length = GROUPS[g][1] >> (t + 1)
                bases[g] = bases[g] - gsides[g][t] * length
                if t > 0:
                    start_ag(g, t - 1)

        for rdma in done:
            rdma.wait_send()

    scratch_shapes = []
    for _, glen in GROUPS:
        for t in range(3):
            scratch_shapes.append(
                pltpu.VMEM((glen >> (t + 1), N), jnp.bfloat16)
            )
    scratch_shapes += [
        pltpu.SemaphoreType.DMA((9,)),
        pltpu.SemaphoreType.DMA((9,)),
        pltpu.SemaphoreType.DMA((9,)),
        pltpu.SemaphoreType.DMA((9,)),
    ]

    return pl.pallas_call(
        body,
        out_shape=jax.ShapeDtypeStruct((M, N), jnp.bfloat16),
        in_specs=[
            pl.BlockSpec(memory_space=pltpu.VMEM),
            pl.BlockSpec(memory_space=pltpu.VMEM),
        ],
        out_specs=pl.BlockSpec(memory_space=pltpu.VMEM),
        scratch_shapes=scratch_shapes,
        compiler_params=pltpu.CompilerParams(collective_id=0),
    )(a16, b16)


# device time: 83167 ns/iter; 2.3072x vs baseline; 1.0585x over previous
import jax
import jax.numpy as jnp
from jax import lax
from jax.experimental import pallas as pl
from jax.experimental.pallas import tpu as pltpu

N_DEV = 8
M = 2048
N = 2048

GROUPS = ((0, 768), (768, 640), (1408, 640))

GMASKS = ((1, 3, 4), (3, 4, 1), (4, 1, 3))


def kernel(A, B):
    a16 = A.astype(jnp.bfloat16)
    b16 = B.astype(jnp.bfloat16)

    def body(a_ref, b_ref, out_ref, *scratch):
        rbufs = [scratch[3 * g : 3 * g + 3] for g in range(3)]
        rs_ssem, rs_rsem, ag_ssem, ag_rsem = scratch[9:]

        my = lax.axis_index("i")
        b0 = my & 1
        b1 = (my >> 1) & 1
        b2 = (my >> 2) & 1
        gsides = (
            (b0 ^ b1, b0, b2),
            (b1, b2, b0),
            (b2, b0 ^ b1, b0),
        )

        barrier_sem = pltpu.get_barrier_semaphore()
        for m in (1, 3, 4):
            pl.semaphore_signal(
                barrier_sem,
                inc=1,
                device_id=(my ^ m,),
                device_id_type=pl.DeviceIdType.MESH,
            )
        pl.semaphore_wait(barrier_sem, 3)

        bases = [None, None, None]
        pend = [None, None, None]
        done = []

        def start_rs(g, t):
            glen = GROUPS[g][1]
            length = glen >> (t + 1)
            side = gsides[g][t]
            keep = bases[g] + side * length
            send = bases[g] + (1 - side) * length
            rdma = pltpu.make_async_remote_copy(
                src_ref=out_ref.at[pl.ds(send, length), :],
                dst_ref=rbufs[g][t],
                send_sem=rs_ssem.at[3 * g + t],
                recv_sem=rs_rsem.at[3 * g + t],
                device_id=(my ^ GMASKS[g][t],),
                device_id_type=pl.DeviceIdType.MESH,
            )
            rdma.start()
            pend[g] = (rdma, keep)

        def start_ag(g, t):
            length = GROUPS[g][1] >> (t + 1)
            rdma = pltpu.make_async_remote_copy(
                src_ref=out_ref.at[pl.ds(bases[g], length), :],
                dst_ref=out_ref.at[pl.ds(bases[g], length), :],
                send_sem=ag_ssem.at[3 * g + t],
                recv_sem=ag_rsem.at[3 * g + t],
                device_id=(my ^ GMASKS[g][t],),
                device_id_type=pl.DeviceIdType.MESH,
            )
            rdma.start()
            pend[g] = rdma

        for g, (off, glen) in enumerate(GROUPS):
            half = glen // 2
            side = gsides[g][0]
            send = off + (1 - side) * half
            keep = off + side * half
            for start in (send, keep):
                out_ref[pl.ds(start, half), :] = jnp.dot(
                    a_ref[pl.ds(start, half), :],
                    b_ref[:, :],
                    preferred_element_type=jnp.float32,
                ).astype(jnp.bfloat16)
                if start is send:
                    bases[g] = off + 0 * my
                    start_rs(g, 0)

        for t in range(3):
            for g in range(3):
                rdma, keep = pend[g]
                rdma.wait_recv()
                done.append(rdma)
                length = GROUPS[g][1] >> (t + 1)
                bases[g] = keep
                if t < 2:
                    nxt = length // 2
                    side = gsides[g][t + 1]
                    first = (1 - side) * nxt
                    second = side * nxt
                    for h in (first, second):
                        out_ref[pl.ds(keep + h, nxt), :] = (
                            out_ref[pl.ds(keep + h, nxt), :]
                            + rbufs[g][t][pl.ds(h, nxt), :]
                        )
                        if h is first:
                            start_rs(g, t + 1)
                else:
                    out_ref[pl.ds(keep, length), :] = (
                        out_ref[pl.ds(keep, length), :] + rbufs[g][t][:, :]
                    )
                    start_ag(g, 2)

        for t in (2, 1, 0):
            for g in range(3):
                rdma = pend[g]
                rdma.wait_recv()
                done.append(rdma)
                length = GROUPS[g][1] >> (t + 1)
                bases[g] = bases[g] - gsides[g][t] * length
                if t > 0:
                    start_ag(g, t - 1)

        for rdma in done:
            rdma.wait_send()

    scratch_shapes = []
    for _, glen in GROUPS:
        for t in range(3):
            scratch_shapes.append(
                pltpu.VMEM((glen >> (t + 1), N), jnp.bfloat16)
            )
    scratch_shapes += [
        pltpu.SemaphoreType.DMA((9,)),
        pltpu.SemaphoreType.DMA((9,)),
        pltpu.SemaphoreType.DMA((9,)),
        pltpu.SemaphoreType.DMA((9,)),
    ]

    return pl.pallas_call(
        body,
        out_shape=jax.ShapeDtypeStruct((M, N), jnp.bfloat16),
        in_specs=[
            pl.BlockSpec(memory_space=pltpu.VMEM),
            pl.BlockSpec(memory_space=pltpu.VMEM),
        ],
        out_specs=pl.BlockSpec(memory_space=pltpu.VMEM),
        scratch_shapes=scratch_shapes,
        compiler_params=pltpu.CompilerParams(collective_id=0),
    )(a16, b16)


# device time: 79332 ns/iter; 2.4187x vs baseline; 1.0483x over previous
import jax
import jax.numpy as jnp
from jax import lax
from jax.experimental import pallas as pl
from jax.experimental.pallas import tpu as pltpu

N_DEV = 8
M = 2048
N = 2048

GROUPS = ((0, 768), (768, 640), (1408, 640))

GMASKS = ((1, 3, 4), (3, 4, 1), (4, 1, 3))


def kernel(A, B):
    a16 = A.astype(jnp.bfloat16)
    b16 = B.astype(jnp.bfloat16)

    def body(a_ref, b_ref, out_ref, *scratch):
        rbufs = [scratch[3 * g : 3 * g + 3] for g in range(3)]
        rs_ssem, rs_rsem, ag_ssem, ag_rsem = scratch[9:]

        my = lax.axis_index("i")

        def side_bits(q):
            qb0 = q & 1
            qb1 = (q >> 1) & 1
            qb2 = (q >> 2) & 1
            return (
                (qb0 ^ qb1, qb0, qb2),
                (qb1, qb2, qb0),
                (qb2, qb0 ^ qb1, qb0),
            )

        gsides = side_bits(my)

        def owned_base(g, q):
            off, glen = GROUPS[g]
            s = side_bits(q)[g]
            return off + s[0] * (glen >> 1) + s[1] * (glen >> 2) + s[2] * (glen >> 3)

        barrier_sem = pltpu.get_barrier_semaphore()
        for m_ in (1, 3, 4):
            pl.semaphore_signal(
                barrier_sem,
                inc=1,
                device_id=(my ^ m_,),
                device_id_type=pl.DeviceIdType.MESH,
            )
        pl.semaphore_wait(barrier_sem, 3)

        bases = [None, None, None]
        pend = [None, None, None]
        done = []

        def rs_idx(g, t, h):
            return 5 * g + 2 * t + h

        def start_rs(g, t):
            glen = GROUPS[g][1]
            length = glen >> (t + 1)
            side = gsides[g][t]
            keep = bases[g] + side * length
            send = bases[g] + (1 - side) * length
            partner = my ^ GMASKS[g][t]
            if t < 2:
                half = length // 2
                side_n = side_bits(partner)[g][t + 1]
                rdmas = []
                for h, x in enumerate(((1 - side_n) * half, side_n * half)):
                    rdma = pltpu.make_async_remote_copy(
                        src_ref=out_ref.at[pl.ds(send + x, half), :],
                        dst_ref=rbufs[g][t].at[pl.ds(x, half), :],
                        send_sem=rs_ssem.at[rs_idx(g, t, h)],
                        recv_sem=rs_rsem.at[rs_idx(g, t, h)],
                        device_id=(partner,),
                        device_id_type=pl.DeviceIdType.MESH,
                    )
                    rdma.start()
                    rdmas.append(rdma)
                pend[g] = (rdmas, keep)
            else:
                rdma = pltpu.make_async_remote_copy(
                    src_ref=out_ref.at[pl.ds(send, length), :],
                    dst_ref=rbufs[g][t],
                    send_sem=rs_ssem.at[rs_idx(g, t, 0)],
                    recv_sem=rs_rsem.at[rs_idx(g, t, 0)],
                    device_id=(partner,),
                    device_id_type=pl.DeviceIdType.MESH,
                )
                rdma.start()
                pend[g] = ([rdma], keep)

        def ag_send(g, piece_origin, to_partner, slot):
            width = GROUPS[g][1] >> 3
            row = owned_base(g, piece_origin)
            rdma = pltpu.make_async_remote_copy(
                src_ref=out_ref.at[pl.ds(row, width), :],
                dst_ref=out_ref.at[pl.ds(row, width), :],
                send_sem=ag_ssem.at[7 * g + slot],
                recv_sem=ag_rsem.at[7 * g + slot],
                device_id=(to_partner,),
                device_id_type=pl.DeviceIdType.MESH,
            )
            rdma.start()
            done.append(rdma)

        def ag_wait(g, slot):
            width = GROUPS[g][1] >> 3
            rdma = pltpu.make_async_remote_copy(
                src_ref=out_ref.at[pl.ds(0, width), :],
                dst_ref=out_ref.at[pl.ds(0, width), :],
                send_sem=ag_ssem.at[7 * g + slot],
                recv_sem=ag_rsem.at[7 * g + slot],
                device_id=(my,),
                device_id_type=pl.DeviceIdType.MESH,
            )
            rdma.wait_recv()

        for g, (off, glen) in enumerate(GROUPS):
            half = glen // 2
            side = gsides[g][0]
            send = off + (1 - side) * half
            keep = off + side * half
            for start in (send, keep):
                out_ref[pl.ds(start, half), :] = jnp.dot(
                    a_ref[pl.ds(start, half), :],
                    b_ref[:, :],
                    preferred_element_type=jnp.float32,
                ).astype(jnp.bfloat16)
                if start is send:
                    bases[g] = off + 0 * my
                    start_rs(g, 0)

        for t in range(3):
            for g in range(3):
                rdmas, keep = pend[g]
                length = GROUPS[g][1] >> (t + 1)
                bases[g] = keep
                if t < 2:
                    half = length // 2
                    side_n = gsides[g][t + 1]
                    xs = ((1 - side_n) * half, side_n * half)
                    for h, x in enumerate(xs):
                        rdmas[h].wait_recv()
                        done.append(rdmas[h])
                        out_ref[pl.ds(keep + x, half), :] = (
                            out_ref[pl.ds(keep + x, half), :]
                            + rbufs[g][t][pl.ds(x, half), :]
                        )
                        if h == 0:
                            start_rs(g, t + 1)
                else:
                    rdmas[0].wait_recv()
                    done.append(rdmas[0])
                    out_ref[pl.ds(keep, length), :] = (
                        out_ref[pl.ds(keep, length), :] + rbufs[g][t][:, :]
                    )
                    m0, m1, m2 = (GMASKS[g][tt] for tt in (0, 1, 2))
                    ag_send(g, my, my ^ m2, 0)
                    ag_send(g, my, my ^ m1, 1)
                    ag_send(g, my, my ^ m0, 3)

        for g in range(3):
            m0, m1, m2 = (GMASKS[g][tt] for tt in (0, 1, 2))
            ag_wait(g, 0)
            ag_send(g, my ^ m2, my ^ m1, 2)
            ag_send(g, my ^ m2, my ^ m0, 4)
        for g in range(3):
            m0, m1, m2 = (GMASKS[g][tt] for tt in (0, 1, 2))
            ag_wait(g, 1)
            ag_send(g, my ^ m1, my ^ m0, 5)
            ag_wait(g, 2)
            ag_send(g, my ^ m1 ^ m2, my ^ m0, 6)
        for g in range(3):
            for slot in (3, 4, 5, 6):
                ag_wait(g, slot)

        for rdma in done:
            rdma.wait_send()

    scratch_shapes = []
    for _, glen in GROUPS:
        for t in range(3):
            scratch_shapes.append(
                pltpu.VMEM((glen >> (t + 1), N), jnp.bfloat16)
            )
    scratch_shapes += [
        pltpu.SemaphoreType.DMA((15,)),
        pltpu.SemaphoreType.DMA((15,)),
        pltpu.SemaphoreType.DMA((21,)),
        pltpu.SemaphoreType.DMA((21,)),
    ]

    return pl.pallas_call(
        body,
        out_shape=jax.ShapeDtypeStruct((M, N), jnp.bfloat16),
        in_specs=[
            pl.BlockSpec(memory_space=pltpu.VMEM),
            pl.BlockSpec(memory_space=pltpu.VMEM),
        ],
        out_specs=pl.BlockSpec(memory_space=pltpu.VMEM),
        scratch_shapes=scratch_shapes,
        compiler_params=pltpu.CompilerParams(collective_id=0),
    )(a16, b16)
